# baseline (speedup 1.0000x reference)
import functools

import jax
import jax.numpy as jnp
from jax.experimental import pallas as pl
from jax.experimental.pallas import tpu as pltpu


def _step_kernel(masks_ref, x_ref, w10_ref, w11_ref, w2_ref, y_ref, last_ref,
                 h_scr, pa_scr, pb_scr, *, Ch, Cx, W, HW, Bblk, N):
    KK = 9
    K0 = 16 + KK * (Cx + Ch)
    K1 = 16 + KK * (Ch + Ch)
    K2 = KK * Ch
    NHALF = 2 if Bblk % 2 == 0 else 1
    IMGS = Bblk // NHALF
    NH = IMGS * HW
    t = pl.program_id(1)

    @pl.when(t == 0)
    def _init():
        h_scr[...] = jnp.zeros_like(h_scr)
        pa_scr[0:16, :] = jnp.zeros((16, N), jnp.bfloat16)
        pa_scr[0:1, :] = jnp.ones((1, N), jnp.bfloat16)

    amounts = []
    for dy in range(3):
        for dx in range(3):
            off = (dy - 1) * W + (dx - 1)
            amounts.append((-off) % NH)

    def build(dst, src, base, stride, lo):
        rows = src.shape[0]
        for i in range(KK):
            amt = amounts[i]
            r = src if amt == 0 else pltpu.roll(src, amt, 1)
            row0 = base + i * stride
            dst[row0:row0 + rows, lo:lo + NH] = r * masks_ref[i:i + 1, 0:NH]

    def gru_layer(cat, h, w1_ref, kcols, w2l, lo):
        build(pa_scr, cat, 16, cat.shape[0], lo)
        acc = jnp.dot(w1_ref[...], pa_scr[0:kcols, lo:lo + NH],
                      preferred_element_type=jnp.float32)
        ru = jax.nn.sigmoid(acc[0:2 * Ch, :])
        reset = ru[0:Ch, :]
        update = ru[Ch:2 * Ch, :]
        hr = cat[cat.shape[0] - Ch:, :] * reset.astype(jnp.bfloat16)
        build(pb_scr, hr, 0, Ch, lo)
        out = jnp.tanh(acc[2 * Ch:3 * Ch, :] +
                       jnp.dot(w2l, pb_scr[0:K2, lo:lo + NH],
                               preferred_element_type=jnp.float32))
        return h + update * (out - h)

    xr = x_ref[...].astype(jnp.bfloat16)

    for half in range(NHALF):
        lo = half * NH
        x = jnp.concatenate(
            [xr[half * IMGS + i] for i in range(IMGS)], axis=1)

        h0 = h_scr[0, :, lo:lo + NH]
        h0n = gru_layer(
            jnp.concatenate([x, h0.astype(jnp.bfloat16)], axis=0),
            h0, w10_ref, K0, w2_ref[0], lo)
        h_scr[0, :, lo:lo + NH] = h0n

        h1 = h_scr[1, :, lo:lo + NH]
        h1n = gru_layer(
            jnp.concatenate(
                [h0n.astype(jnp.bfloat16), h1.astype(jnp.bfloat16)], axis=0),
            h1, w11_ref, K1, w2_ref[1], lo)
        h_scr[1, :, lo:lo + NH] = h1n

        h1b = h1n.astype(jnp.bfloat16)
        for i in range(IMGS):
            y_ref[half * IMGS + i, :, :] = h1b[:, i * HW:(i + 1) * HW]

    @pl.when(t == pl.num_programs(1) - 1)
    def _emit_last():
        for l in range(2):
            hl = h_scr[l].astype(jnp.bfloat16)
            for bb in range(Bblk):
                last_ref[bb, l, :, :] = hl[:, bb * HW:(bb + 1) * HW]


def _convgru(xr, masks, w10, w11, w2b, *, Ch, Cx, W, HW, Bblk, N):
    T, NB = xr.shape[0], xr.shape[1]
    KK = 9
    K0 = 16 + KK * (Cx + Ch)
    K1 = 16 + KK * (Ch + Ch)
    body = functools.partial(_step_kernel, Ch=Ch, Cx=Cx, W=W, HW=HW,
                             Bblk=Bblk, N=N)
    return pl.pallas_call(
        body,
        out_shape=(
            jax.ShapeDtypeStruct((T, NB * Bblk, Ch, HW), jnp.bfloat16),
            jax.ShapeDtypeStruct((NB * Bblk, 2, Ch, HW), jnp.bfloat16),
        ),
        grid=(NB, T),
        in_specs=[
            pl.BlockSpec((16, N), lambda b, t: (0, 0)),
            pl.BlockSpec((None, None, Bblk, Cx, HW),
                         lambda b, t: (t, b, 0, 0, 0)),
            pl.BlockSpec((3 * Ch, K0), lambda b, t: (0, 0)),
            pl.BlockSpec((3 * Ch, K1), lambda b, t: (0, 0)),
            pl.BlockSpec((2, Ch, KK * Ch), lambda b, t: (0, 0, 0)),
        ],
        out_specs=(
            pl.BlockSpec((None, Bblk, Ch, HW), lambda b, t: (t, b, 0, 0)),
            pl.BlockSpec((Bblk, 2, Ch, HW), lambda b, t: (b, 0, 0, 0)),
        ),
        scratch_shapes=[
            pltpu.VMEM((2, Ch, N), jnp.float32),
            pltpu.VMEM((K1, N), jnp.bfloat16),
            pltpu.VMEM((KK * Ch, N), jnp.bfloat16),
        ],
        compiler_params=pltpu.CompilerParams(
            dimension_semantics=("parallel", "arbitrary")),
    )(masks, xr, w10, w11, w2b)


def kernel(x, w1_all, w2_all):
    T, B, Cx, H, W = x.shape
    Ch = w2_all.shape[1]
    NB = 2
    Bblk = B // NB
    HW = H * W
    N = Bblk * HW
    KK = 9
    SX = 64
    S = 128

    def repack_w1(w, cx):
        taps = []
        for i in range(KK):
            xcols = w[:, i * S:i * S + cx]
            hcols = w[:, i * S + SX:i * S + SX + Ch]
            taps.append(jnp.concatenate([xcols, hcols], axis=1))
        bias = w[:, KK * S:KK * S + 1]
        pad = jnp.zeros((w.shape[0], 15), w.dtype)
        return jnp.concatenate([bias, pad] + taps, axis=1).astype(jnp.bfloat16)

    w10 = repack_w1(w1_all[0], Cx)
    w11 = repack_w1(w1_all[1], Ch)
    w2b = w2_all.astype(jnp.bfloat16)

    pos = jnp.arange(HW)
    row, col = pos // W, pos % W
    mrows = []
    for dy in range(3):
        for dx in range(3):
            oy, ox = dy - 1, dx - 1
            mrows.append((row + oy >= 0) & (row + oy < H) &
                         (col + ox >= 0) & (col + ox < W))
    masks = jnp.tile(jnp.stack(mrows), (1, Bblk))
    masks = jnp.concatenate(
        [masks, jnp.zeros((16 - KK, N), masks.dtype)], axis=0)
    masks = masks.astype(jnp.bfloat16)

    xr = x.reshape(T, NB, Bblk, Cx, HW)

    y_seq, last = _convgru(xr, masks, w10, w11, w2b,
                           Ch=Ch, Cx=Cx, W=W, HW=HW, Bblk=Bblk, N=N)

    return (y_seq.astype(jnp.float32).reshape(T, B, Ch, H, W),
            last.astype(jnp.float32).reshape(B, 2, Ch, H, W))


# speedup vs baseline: 1.1462x; 1.1462x over previous
import functools

import jax
import jax.numpy as jnp
from jax.experimental import pallas as pl
from jax.experimental.pallas import tpu as pltpu


def _step_kernel(masks_ref, x_ref, w10_ref, w11_ref, w2_ref, y_ref, last_ref,
                 h_scr, pa_scr, pb_scr, *, Ch, Cx, W, HW, Bblk, N):
    KK = 9
    K0 = 16 + KK * (Cx + Ch)
    K1 = 16 + KK * (Ch + Ch)
    K2 = KK * Ch
    t = pl.program_id(1)

    @pl.when(t == 0)
    def _init():
        h_scr[...] = jnp.zeros_like(h_scr)
        pa_scr[0:16, :] = jnp.zeros((16, N), jnp.bfloat16)
        pa_scr[0:1, :] = jnp.ones((1, N), jnp.bfloat16)

    amounts = []
    for dy in range(3):
        for dx in range(3):
            off = (dy - 1) * W + (dx - 1)
            amounts.append((-off) % N)

    def build(dst, src, base, stride):
        rows = src.shape[0]
        for i in range(KK):
            amt = amounts[i]
            r = src if amt == 0 else pltpu.roll(src, amt, 1)
            row0 = base + i * stride
            dst[row0:row0 + rows, :] = r * masks_ref[i:i + 1, :]

    def gru_layer(cat, h, w1_ref, kcols, w2l):
        build(pa_scr, cat, 16, cat.shape[0])
        acc = jnp.dot(w1_ref[...], pa_scr[0:kcols, :],
                      preferred_element_type=jnp.float32)
        ru = jax.nn.sigmoid(acc[0:2 * Ch, :])
        reset = ru[0:Ch, :]
        update = ru[Ch:2 * Ch, :]
        hr = cat[cat.shape[0] - Ch:, :] * reset.astype(jnp.bfloat16)
        build(pb_scr, hr, 0, Ch)
        out = jnp.tanh(acc[2 * Ch:3 * Ch, :] +
                       jnp.dot(w2l, pb_scr[0:K2, :],
                               preferred_element_type=jnp.float32))
        return h + update * (out - h)

    xr = x_ref[...].astype(jnp.bfloat16)
    x = jnp.concatenate([xr[bb] for bb in range(Bblk)], axis=1)

    h0 = h_scr[0]
    h0n = gru_layer(jnp.concatenate([x, h0.astype(jnp.bfloat16)], axis=0),
                    h0, w10_ref, K0, w2_ref[0])
    h_scr[0] = h0n

    h1 = h_scr[1]
    h1n = gru_layer(
        jnp.concatenate([h0n.astype(jnp.bfloat16), h1.astype(jnp.bfloat16)],
                        axis=0),
        h1, w11_ref, K1, w2_ref[1])
    h_scr[1] = h1n

    h1b = h1n.astype(jnp.bfloat16)
    for bb in range(Bblk):
        y_ref[bb, :, :] = h1b[:, bb * HW:(bb + 1) * HW]

    @pl.when(t == pl.num_programs(1) - 1)
    def _emit_last():
        for l in range(2):
            hl = h_scr[l].astype(jnp.bfloat16)
            for bb in range(Bblk):
                last_ref[bb, l, :, :] = hl[:, bb * HW:(bb + 1) * HW]


def _convgru(xr, masks, w10, w11, w2b, *, Ch, Cx, W, HW, Bblk, N):
    T, NB = xr.shape[0], xr.shape[1]
    KK = 9
    K0 = 16 + KK * (Cx + Ch)
    K1 = 16 + KK * (Ch + Ch)
    body = functools.partial(_step_kernel, Ch=Ch, Cx=Cx, W=W, HW=HW,
                             Bblk=Bblk, N=N)
    return pl.pallas_call(
        body,
        out_shape=(
            jax.ShapeDtypeStruct((T, NB * Bblk, Ch, HW), jnp.bfloat16),
            jax.ShapeDtypeStruct((NB * Bblk, 2, Ch, HW), jnp.bfloat16),
        ),
        grid=(NB, T),
        in_specs=[
            pl.BlockSpec((16, N), lambda b, t: (0, 0)),
            pl.BlockSpec((None, None, Bblk, Cx, HW),
                         lambda b, t: (t, b, 0, 0, 0)),
            pl.BlockSpec((3 * Ch, K0), lambda b, t: (0, 0)),
            pl.BlockSpec((3 * Ch, K1), lambda b, t: (0, 0)),
            pl.BlockSpec((2, Ch, KK * Ch), lambda b, t: (0, 0, 0)),
        ],
        out_specs=(
            pl.BlockSpec((None, Bblk, Ch, HW), lambda b, t: (t, b, 0, 0)),
            pl.BlockSpec((Bblk, 2, Ch, HW), lambda b, t: (b, 0, 0, 0)),
        ),
        scratch_shapes=[
            pltpu.VMEM((2, Ch, N), jnp.float32),
            pltpu.VMEM((K1, N), jnp.bfloat16),
            pltpu.VMEM((KK * Ch, N), jnp.bfloat16),
        ],
        compiler_params=pltpu.CompilerParams(
            dimension_semantics=("parallel", "arbitrary")),
    )(masks, xr, w10, w11, w2b)


def kernel(x, w1_all, w2_all):
    T, B, Cx, H, W = x.shape
    Ch = w2_all.shape[1]
    NB = 2
    Bblk = B // NB
    HW = H * W
    N = Bblk * HW
    KK = 9
    SX = 64
    S = 128

    def repack_w1(w, cx):
        taps = []
        for i in range(KK):
            xcols = w[:, i * S:i * S + cx]
            hcols = w[:, i * S + SX:i * S + SX + Ch]
            taps.append(jnp.concatenate([xcols, hcols], axis=1))
        bias = w[:, KK * S:KK * S + 1]
        pad = jnp.zeros((w.shape[0], 15), w.dtype)
        return jnp.concatenate([bias, pad] + taps, axis=1).astype(jnp.bfloat16)

    w10 = repack_w1(w1_all[0], Cx)
    w11 = repack_w1(w1_all[1], Ch)
    w2b = w2_all.astype(jnp.bfloat16)

    pos = jnp.arange(HW)
    row, col = pos // W, pos % W
    mrows = []
    for dy in range(3):
        for dx in range(3):
            oy, ox = dy - 1, dx - 1
            mrows.append((row + oy >= 0) & (row + oy < H) &
                         (col + ox >= 0) & (col + ox < W))
    masks = jnp.tile(jnp.stack(mrows), (1, Bblk))
    masks = jnp.concatenate(
        [masks, jnp.zeros((16 - KK, N), masks.dtype)], axis=0)
    masks = masks.astype(jnp.bfloat16)

    xr = x.reshape(T, NB, Bblk, Cx, HW)

    y_seq, last = _convgru(xr, masks, w10, w11, w2b,
                           Ch=Ch, Cx=Cx, W=W, HW=HW, Bblk=Bblk, N=N)

    return (y_seq.astype(jnp.float32).reshape(T, B, Ch, H, W),
            last.astype(jnp.float32).reshape(B, 2, Ch, H, W))


# speedup vs baseline: 1.1538x; 1.0067x over previous
import functools

import jax
import jax.numpy as jnp
from jax.experimental import pallas as pl
from jax.experimental.pallas import tpu as pltpu


def _step_kernel(masks_ref, x_ref, w10_ref, w11_ref, w2_ref, y_ref, last_ref,
                 h_scr, pa_scr, pb_scr, *, Ch, Cx, W, HW, Bblk, N):
    KK = 9
    K0 = 16 + KK * (Cx + Ch)
    K1 = 16 + KK * (Ch + Ch)
    K2 = KK * Ch
    t = pl.program_id(1)

    @pl.when(t == 0)
    def _init():
        h_scr[...] = jnp.zeros_like(h_scr)
        pa_scr[0:16, :] = jnp.zeros((16, N), jnp.bfloat16)
        pa_scr[0:1, :] = jnp.ones((1, N), jnp.bfloat16)

    amounts = []
    for dy in range(3):
        for dx in range(3):
            off = (dy - 1) * W + (dx - 1)
            amounts.append((-off) % N)

    def build(dst, src, base, stride):
        rows = src.shape[0]
        for i in range(KK):
            amt = amounts[i]
            r = src if amt == 0 else pltpu.roll(src, amt, 1)
            row0 = base + i * stride
            if i == 4:
                dst[row0:row0 + rows, :] = r
            else:
                dst[row0:row0 + rows, :] = r * masks_ref[i:i + 1, :]

    def gru_layer(cat, h, w1_ref, kcols, w2l):
        build(pa_scr, cat, 16, cat.shape[0])
        acc = jnp.dot(w1_ref[...], pa_scr[0:kcols, :],
                      preferred_element_type=jnp.float32)
        ru = 0.5 + 0.5 * jnp.tanh(0.5 * acc[0:2 * Ch, :])
        reset = ru[0:Ch, :]
        update = ru[Ch:2 * Ch, :]
        hr = cat[cat.shape[0] - Ch:, :] * reset.astype(jnp.bfloat16)
        build(pb_scr, hr, 0, Ch)
        out = jnp.tanh(acc[2 * Ch:3 * Ch, :] +
                       jnp.dot(w2l, pb_scr[0:K2, :],
                               preferred_element_type=jnp.float32))
        return h + update * (out - h)

    xr = x_ref[...].astype(jnp.bfloat16)
    x = jnp.concatenate([xr[bb] for bb in range(Bblk)], axis=1)

    h0 = h_scr[0]
    h0n = gru_layer(jnp.concatenate([x, h0.astype(jnp.bfloat16)], axis=0),
                    h0, w10_ref, K0, w2_ref[0])
    h_scr[0] = h0n

    h1 = h_scr[1]
    h1n = gru_layer(
        jnp.concatenate([h0n.astype(jnp.bfloat16), h1.astype(jnp.bfloat16)],
                        axis=0),
        h1, w11_ref, K1, w2_ref[1])
    h_scr[1] = h1n

    h1b = h1n.astype(jnp.bfloat16)
    for bb in range(Bblk):
        y_ref[bb, :, :] = h1b[:, bb * HW:(bb + 1) * HW]

    @pl.when(t == pl.num_programs(1) - 1)
    def _emit_last():
        for l in range(2):
            hl = h_scr[l].astype(jnp.bfloat16)
            for bb in range(Bblk):
                last_ref[bb, l, :, :] = hl[:, bb * HW:(bb + 1) * HW]


def _convgru(xr, masks, w10, w11, w2b, *, Ch, Cx, W, HW, Bblk, N):
    T, NB = xr.shape[0], xr.shape[1]
    KK = 9
    K0 = 16 + KK * (Cx + Ch)
    K1 = 16 + KK * (Ch + Ch)
    body = functools.partial(_step_kernel, Ch=Ch, Cx=Cx, W=W, HW=HW,
                             Bblk=Bblk, N=N)
    return pl.pallas_call(
        body,
        out_shape=(
            jax.ShapeDtypeStruct((T, NB * Bblk, Ch, HW), jnp.bfloat16),
            jax.ShapeDtypeStruct((NB * Bblk, 2, Ch, HW), jnp.bfloat16),
        ),
        grid=(NB, T),
        in_specs=[
            pl.BlockSpec((16, N), lambda b, t: (0, 0)),
            pl.BlockSpec((None, None, Bblk, Cx, HW),
                         lambda b, t: (t, b, 0, 0, 0)),
            pl.BlockSpec((3 * Ch, K0), lambda b, t: (0, 0)),
            pl.BlockSpec((3 * Ch, K1), lambda b, t: (0, 0)),
            pl.BlockSpec((2, Ch, KK * Ch), lambda b, t: (0, 0, 0)),
        ],
        out_specs=(
            pl.BlockSpec((None, Bblk, Ch, HW), lambda b, t: (t, b, 0, 0)),
            pl.BlockSpec((Bblk, 2, Ch, HW), lambda b, t: (b, 0, 0, 0)),
        ),
        scratch_shapes=[
            pltpu.VMEM((2, Ch, N), jnp.float32),
            pltpu.VMEM((K1, N), jnp.bfloat16),
            pltpu.VMEM((KK * Ch, N), jnp.bfloat16),
        ],
        compiler_params=pltpu.CompilerParams(
            dimension_semantics=("parallel", "arbitrary")),
    )(masks, xr, w10, w11, w2b)


def kernel(x, w1_all, w2_all):
    T, B, Cx, H, W = x.shape
    Ch = w2_all.shape[1]
    NB = 2
    Bblk = B // NB
    HW = H * W
    N = Bblk * HW
    KK = 9
    SX = 64
    S = 128

    def repack_w1(w, cx):
        taps = []
        for i in range(KK):
            xcols = w[:, i * S:i * S + cx]
            hcols = w[:, i * S + SX:i * S + SX + Ch]
            taps.append(jnp.concatenate([xcols, hcols], axis=1))
        bias = w[:, KK * S:KK * S + 1]
        pad = jnp.zeros((w.shape[0], 15), w.dtype)
        return jnp.concatenate([bias, pad] + taps, axis=1).astype(jnp.bfloat16)

    w10 = repack_w1(w1_all[0], Cx)
    w11 = repack_w1(w1_all[1], Ch)
    w2b = w2_all.astype(jnp.bfloat16)

    pos = jnp.arange(HW)
    row, col = pos // W, pos % W
    mrows = []
    for dy in range(3):
        for dx in range(3):
            oy, ox = dy - 1, dx - 1
            mrows.append((row + oy >= 0) & (row + oy < H) &
                         (col + ox >= 0) & (col + ox < W))
    masks = jnp.tile(jnp.stack(mrows), (1, Bblk))
    masks = jnp.concatenate(
        [masks, jnp.zeros((16 - KK, N), masks.dtype)], axis=0)
    masks = masks.astype(jnp.bfloat16)

    xr = x.reshape(T, NB, Bblk, Cx, HW)

    y_seq, last = _convgru(xr, masks, w10, w11, w2b,
                           Ch=Ch, Cx=Cx, W=W, HW=HW, Bblk=Bblk, N=N)

    return (y_seq.astype(jnp.float32).reshape(T, B, Ch, H, W),
            last.astype(jnp.float32).reshape(B, 2, Ch, H, W))
